# baseline (speedup 1.0000x reference)
import functools

import jax
import jax.numpy as jnp
from jax import lax
from jax.experimental import pallas as pl
from jax.experimental.pallas import tpu as pltpu
from jax.experimental.pallas import tpu_sc as plsc

_LANES = 16
_VB = 1024


def _pack_body(a_ref, b_ref, o_ref):
    emb_d = a_ref.shape[0]
    slot_w = o_ref.shape[1]
    di = lax.broadcasted_iota(jnp.int32, (emb_d, slot_w), 0)
    ji = lax.broadcasted_iota(jnp.int32, (emb_d, slot_w), 1)
    itop = (di == ji).astype(jnp.float32)
    ibot = (di + emb_d == ji).astype(jnp.float32)
    dn = (((0,), (0,)), ((), ()))
    o_ref[...] = (
        lax.dot_general(a_ref[...], itop, dn, preferred_element_type=jnp.float32)
        + lax.dot_general(b_ref[...], ibot, dn, preferred_element_type=jnp.float32)
    )


@functools.lru_cache(maxsize=None)
def _tc_pack_manual(emb_d, vocab, slot_w):
    split = (vocab // 2 + _VB - 1) // _VB * _VB
    nblk = split // _VB
    vocab_al = vocab // 128 * 128
    tail_n = vocab - vocab_al
    tail_lo = (vocab_al - split) - (nblk - 1) * _VB

    nbuf = 3

    def body(t_hbm, tail_ref, o_hbm, ab_buf, o_buf, sab, so):
        di = lax.broadcasted_iota(jnp.int32, (emb_d, slot_w), 0)
        ji = lax.broadcasted_iota(jnp.int32, (emb_d, slot_w), 1)
        itop = (di == ji).astype(jnp.float32)
        ibot = (di + emb_d == ji).astype(jnp.float32)
        dn = (((0,), (0,)), ((), ()))

        def bsize(g):
            return min(vocab_al - (split + g * _VB), _VB)

        def start_in(g, slot):
            pltpu.make_async_copy(
                t_hbm.at[:, pl.ds(g * _VB, _VB)],
                ab_buf.at[slot, pl.ds(0, emb_d), :], sab.at[slot],
            ).start()
            w = bsize(g)
            pltpu.make_async_copy(
                t_hbm.at[:, pl.ds(split + g * _VB, w)],
                ab_buf.at[slot, pl.ds(emb_d, emb_d), pl.ds(0, w)],
                sab.at[slot],
            ).start()

        def wait_in(g, slot):
            pltpu.make_async_copy(
                t_hbm.at[:, pl.ds(0, _VB)],
                ab_buf.at[slot, pl.ds(0, emb_d), :], sab.at[slot]).wait()
            w = bsize(g)
            pltpu.make_async_copy(
                t_hbm.at[:, pl.ds(0, w)],
                ab_buf.at[slot, pl.ds(0, emb_d), pl.ds(0, w)],
                sab.at[slot]).wait()

        def out_copy(g, slot):
            return pltpu.make_async_copy(
                o_buf.at[slot], o_hbm.at[pl.ds(g * _VB, _VB), :], so.at[slot])

        for g in range(min(nbuf, nblk)):
            start_in(g, g % nbuf)
        for g in range(nblk):
            slot = g % nbuf
            wait_in(g, slot)
            if g >= nbuf:
                out_copy(g - nbuf, slot).wait()
            o_buf[slot] = (
                lax.dot_general(ab_buf[slot, pl.ds(0, emb_d), :], itop, dn,
                                preferred_element_type=jnp.float32)
                + lax.dot_general(ab_buf[slot, pl.ds(emb_d, emb_d), :], ibot,
                                  dn, preferred_element_type=jnp.float32)
            )
            if g == nblk - 1 and tail_n:
                o_buf[slot, pl.ds(tail_lo, tail_n), pl.ds(emb_d, emb_d)] = (
                    tail_ref[...])
            out_copy(g, slot).start()
            if g + nbuf < nblk:
                start_in(g + nbuf, slot)
        for g in range(max(nblk - nbuf, 0), nblk):
            out_copy(g, g % nbuf).wait()

    return pl.pallas_call(
        body,
        in_specs=[pl.BlockSpec(memory_space=pl.ANY), pl.BlockSpec()],
        out_specs=pl.BlockSpec(memory_space=pl.ANY),
        out_shape=jax.ShapeDtypeStruct((split, slot_w), jnp.float32),
        scratch_shapes=[
            pltpu.VMEM((nbuf, slot_w, _VB), jnp.float32),
            pltpu.VMEM((nbuf, _VB, slot_w), jnp.float32),
            pltpu.SemaphoreType.DMA((nbuf,)),
            pltpu.SemaphoreType.DMA((nbuf,)),
        ],
    ), split


@functools.lru_cache(maxsize=None)
def _tc_pack(emb_d, vocab, slot_w):
    split = (vocab // 2 + _VB - 1) // _VB * _VB
    grid = split // _VB
    nblk = split // _VB

    return pl.pallas_call(
        _pack_body,
        grid=(grid,),
        in_specs=[
            pl.BlockSpec((emb_d, _VB), lambda i: (0, i)),
            pl.BlockSpec((emb_d, _VB), lambda i, n=nblk: (0, i + n)),
        ],
        out_specs=pl.BlockSpec((_VB, slot_w), lambda i: (i, 0)),
        out_shape=jax.ShapeDtypeStruct((split, slot_w), jnp.float32),
        compiler_params=pltpu.CompilerParams(
            dimension_semantics=("parallel",)),
    ), split


@functools.lru_cache(maxsize=None)
def _sc_partial_dot(n_slots, slot_w, batch, emb_d, split):
    info = plsc.get_sparse_core_info()
    nc, ns = info.num_cores, info.num_subcores
    nw = nc * ns
    assert batch % nw == 0
    b_per_w = batch // nw
    assert b_per_w <= 128
    assert emb_d % _LANES == 0
    chunks = emb_d // _LANES

    mesh = plsc.VectorSubcoreMesh(core_axis_name="c", subcore_axis_name="s")

    @functools.partial(
        pl.kernel,
        out_type=jax.ShapeDtypeStruct((nw, _LANES), jnp.float32),
        mesh=mesh,
        scratch_types=[
            pltpu.VMEM((b_per_w,), jnp.int32),
            pltpu.VMEM((b_per_w,), jnp.int32),
            pltpu.VMEM((b_per_w,), jnp.int32),
            pltpu.VMEM((b_per_w,), jnp.int32),
            pltpu.VMEM((b_per_w, slot_w), jnp.float32),
            pltpu.VMEM((b_per_w, slot_w), jnp.float32),
            pltpu.VMEM((_LANES,), jnp.float32),
            pltpu.SemaphoreType.DMA,
        ],
    )
    def sc_kernel(focus_hbm, context_hbm, emb_hbm, out_hbm,
                  idx_f, idx_c, slot_f, slot_c, rows_f, rows_c, acc_v, sem):
        wid = lax.axis_index("s") * nc + lax.axis_index("c")
        base = wid * b_per_w
        pltpu.sync_copy(focus_hbm.at[pl.ds(base, b_per_w)], idx_f)
        pltpu.sync_copy(context_hbm.at[pl.ds(base, b_per_w)], idx_c)
        for k in range(b_per_w // _LANES):
            sl = pl.ds(k * _LANES, _LANES)
            vf = idx_f[sl]
            vc = idx_c[sl]
            slot_f[sl] = jnp.where(vf >= split, vf - split, vf)
            slot_c[sl] = jnp.where(vc >= split, vc - split, vc)
        cp_f = pltpu.async_copy(emb_hbm.at[slot_f], rows_f, sem)
        cp_c = pltpu.async_copy(emb_hbm.at[slot_c], rows_c, sem)
        cp_f.wait()
        cp_c.wait()

        def body(k, acc):
            base_k = k * _LANES
            vf = idx_f[pl.ds(base_k, _LANES)]
            vc = idx_c[pl.ds(base_k, _LANES)]
            pf = jnp.where(vf >= split, emb_d, 0)
            pc = jnp.where(vc >= split, emb_d, 0)
            for r in range(_LANES):
                i = base_k + r
                f_off = pf[r]
                c_off = pc[r]
                for j in range(chunks):
                    f = rows_f[i, pl.ds(f_off + j * _LANES, _LANES)]
                    c = rows_c[i, pl.ds(c_off + j * _LANES, _LANES)]
                    acc = acc + f * c
            return acc

        acc = lax.fori_loop(0, b_per_w // _LANES, body,
                            jnp.zeros((_LANES,), jnp.float32))
        acc_v[...] = acc
        pltpu.sync_copy(acc_v, out_hbm.at[wid])

    return sc_kernel


def _tc_finish_body(p_ref, o_ref):
    s = jnp.sum(p_ref[...])
    val = jnp.minimum(s, 0.0) - jnp.log(1.0 + jnp.exp(-jnp.abs(s)))
    o_ref[...] = jnp.broadcast_to(val, (1, 1))


_tc_finish = pl.pallas_call(
    _tc_finish_body,
    out_shape=jax.ShapeDtypeStruct((1, 1), jnp.float32),
)


def kernel(focus, context, embeddings):
    focus = focus.astype(jnp.int32)
    context = context.astype(jnp.int32)
    vocab, emb_d = embeddings.shape
    slot_w = 128
    emb_t = embeddings.T
    pack, split = _tc_pack_manual(emb_d, vocab, slot_w)
    vocab_al = vocab // 128 * 128
    emb_slots = pack(emb_t, embeddings[vocab_al:vocab, :])
    partials = _sc_partial_dot(
        emb_slots.shape[0], slot_w, focus.shape[0], emb_d, split)(
        focus, context, emb_slots)
    return _tc_finish(partials)


# speedup vs baseline: 1.0598x; 1.0598x over previous
import functools

import jax
import jax.numpy as jnp
from jax import lax
from jax.experimental import pallas as pl
from jax.experimental.pallas import tpu as pltpu
from jax.experimental.pallas import tpu_sc as plsc

_LANES = 16
_VB = 1024


def _pack_body(a_ref, b_ref, o_ref):
    emb_d = a_ref.shape[0]
    slot_w = o_ref.shape[1]
    di = lax.broadcasted_iota(jnp.int32, (emb_d, slot_w), 0)
    ji = lax.broadcasted_iota(jnp.int32, (emb_d, slot_w), 1)
    itop = (di == ji).astype(jnp.float32)
    ibot = (di + emb_d == ji).astype(jnp.float32)
    dn = (((0,), (0,)), ((), ()))
    o_ref[...] = (
        lax.dot_general(a_ref[...], itop, dn, preferred_element_type=jnp.float32)
        + lax.dot_general(b_ref[...], ibot, dn, preferred_element_type=jnp.float32)
    )


@functools.lru_cache(maxsize=None)
def _tc_pack_manual(emb_d, vocab, slot_w):
    split = (vocab // 2 + _VB - 1) // _VB * _VB
    nblk = split // _VB
    vocab_al = vocab // 128 * 128
    tail_n = vocab - vocab_al
    tail_lo = (vocab_al - split) - (nblk - 1) * _VB

    nbuf = 4

    def body(t_hbm, tail_ref, o_hbm, ab_buf, o_buf, sab, so):
        di = lax.broadcasted_iota(jnp.int32, (emb_d, slot_w), 0)
        ji = lax.broadcasted_iota(jnp.int32, (emb_d, slot_w), 1)
        itop = (di == ji).astype(jnp.float32)
        ibot = (di + emb_d == ji).astype(jnp.float32)
        dn = (((0,), (0,)), ((), ()))

        def bsize(g):
            return min(vocab_al - (split + g * _VB), _VB)

        def start_in(g, slot):
            pltpu.make_async_copy(
                t_hbm.at[:, pl.ds(g * _VB, _VB)],
                ab_buf.at[slot, pl.ds(0, emb_d), :], sab.at[slot],
            ).start()
            w = bsize(g)
            pltpu.make_async_copy(
                t_hbm.at[:, pl.ds(split + g * _VB, w)],
                ab_buf.at[slot, pl.ds(emb_d, emb_d), pl.ds(0, w)],
                sab.at[slot],
            ).start()

        def wait_in(g, slot):
            pltpu.make_async_copy(
                t_hbm.at[:, pl.ds(0, _VB)],
                ab_buf.at[slot, pl.ds(0, emb_d), :], sab.at[slot]).wait()
            w = bsize(g)
            pltpu.make_async_copy(
                t_hbm.at[:, pl.ds(0, w)],
                ab_buf.at[slot, pl.ds(0, emb_d), pl.ds(0, w)],
                sab.at[slot]).wait()

        def out_copy(g, slot):
            return pltpu.make_async_copy(
                o_buf.at[slot], o_hbm.at[pl.ds(g * _VB, _VB), :], so.at[slot])

        for g in range(min(nbuf, nblk)):
            start_in(g, g % nbuf)
        for g in range(nblk):
            slot = g % nbuf
            wait_in(g, slot)
            if g >= nbuf:
                out_copy(g - nbuf, slot).wait()
            o_buf[slot] = (
                lax.dot_general(ab_buf[slot, pl.ds(0, emb_d), :], itop, dn,
                                preferred_element_type=jnp.float32)
                + lax.dot_general(ab_buf[slot, pl.ds(emb_d, emb_d), :], ibot,
                                  dn, preferred_element_type=jnp.float32)
            )
            if g == nblk - 1 and tail_n:
                o_buf[slot, pl.ds(tail_lo, tail_n), pl.ds(emb_d, emb_d)] = (
                    tail_ref[...])
            out_copy(g, slot).start()
            if g + nbuf < nblk:
                start_in(g + nbuf, slot)
        for g in range(max(nblk - nbuf, 0), nblk):
            out_copy(g, g % nbuf).wait()

    return pl.pallas_call(
        body,
        in_specs=[pl.BlockSpec(memory_space=pl.ANY), pl.BlockSpec()],
        out_specs=pl.BlockSpec(memory_space=pl.ANY),
        out_shape=jax.ShapeDtypeStruct((split, slot_w), jnp.float32),
        scratch_shapes=[
            pltpu.VMEM((nbuf, slot_w, _VB), jnp.float32),
            pltpu.VMEM((nbuf, _VB, slot_w), jnp.float32),
            pltpu.SemaphoreType.DMA((nbuf,)),
            pltpu.SemaphoreType.DMA((nbuf,)),
        ],
    ), split


@functools.lru_cache(maxsize=None)
def _tc_pack(emb_d, vocab, slot_w):
    split = (vocab // 2 + _VB - 1) // _VB * _VB
    grid = split // _VB
    nblk = split // _VB

    return pl.pallas_call(
        _pack_body,
        grid=(grid,),
        in_specs=[
            pl.BlockSpec((emb_d, _VB), lambda i: (0, i)),
            pl.BlockSpec((emb_d, _VB), lambda i, n=nblk: (0, i + n)),
        ],
        out_specs=pl.BlockSpec((_VB, slot_w), lambda i: (i, 0)),
        out_shape=jax.ShapeDtypeStruct((split, slot_w), jnp.float32),
        compiler_params=pltpu.CompilerParams(
            dimension_semantics=("parallel",)),
    ), split


@functools.lru_cache(maxsize=None)
def _sc_partial_dot(n_slots, slot_w, batch, emb_d, split):
    info = plsc.get_sparse_core_info()
    nc, ns = info.num_cores, info.num_subcores
    nw = nc * ns
    assert batch % nw == 0
    b_per_w = batch // nw
    assert b_per_w <= 128
    assert emb_d % _LANES == 0
    chunks = emb_d // _LANES

    mesh = plsc.VectorSubcoreMesh(core_axis_name="c", subcore_axis_name="s")

    @functools.partial(
        pl.kernel,
        out_type=jax.ShapeDtypeStruct((nw, _LANES), jnp.float32),
        mesh=mesh,
        scratch_types=[
            pltpu.VMEM((b_per_w,), jnp.int32),
            pltpu.VMEM((b_per_w,), jnp.int32),
            pltpu.VMEM((b_per_w,), jnp.int32),
            pltpu.VMEM((b_per_w,), jnp.int32),
            pltpu.VMEM((b_per_w, slot_w), jnp.float32),
            pltpu.VMEM((b_per_w, slot_w), jnp.float32),
            pltpu.VMEM((_LANES,), jnp.float32),
            pltpu.SemaphoreType.DMA,
        ],
    )
    def sc_kernel(focus_hbm, context_hbm, emb_hbm, out_hbm,
                  idx_f, idx_c, slot_f, slot_c, rows_f, rows_c, acc_v, sem):
        wid = lax.axis_index("s") * nc + lax.axis_index("c")
        base = wid * b_per_w
        pltpu.sync_copy(focus_hbm.at[pl.ds(base, b_per_w)], idx_f)
        pltpu.sync_copy(context_hbm.at[pl.ds(base, b_per_w)], idx_c)
        for k in range(b_per_w // _LANES):
            sl = pl.ds(k * _LANES, _LANES)
            vf = idx_f[sl]
            vc = idx_c[sl]
            slot_f[sl] = jnp.where(vf >= split, vf - split, vf)
            slot_c[sl] = jnp.where(vc >= split, vc - split, vc)
        cp_f = pltpu.async_copy(emb_hbm.at[slot_f], rows_f, sem)
        cp_c = pltpu.async_copy(emb_hbm.at[slot_c], rows_c, sem)
        cp_f.wait()
        cp_c.wait()

        def body(k, acc):
            base_k = k * _LANES
            vf = idx_f[pl.ds(base_k, _LANES)]
            vc = idx_c[pl.ds(base_k, _LANES)]
            pf = jnp.where(vf >= split, emb_d, 0)
            pc = jnp.where(vc >= split, emb_d, 0)
            for r in range(_LANES):
                i = base_k + r
                f_off = pf[r]
                c_off = pc[r]
                for j in range(chunks):
                    f = rows_f[i, pl.ds(f_off + j * _LANES, _LANES)]
                    c = rows_c[i, pl.ds(c_off + j * _LANES, _LANES)]
                    acc = acc + f * c
            return acc

        acc = lax.fori_loop(0, b_per_w // _LANES, body,
                            jnp.zeros((_LANES,), jnp.float32))
        acc_v[...] = acc
        pltpu.sync_copy(acc_v, out_hbm.at[wid])

    return sc_kernel


def _tc_finish_body(p_ref, o_ref):
    s = jnp.sum(p_ref[...])
    val = jnp.minimum(s, 0.0) - jnp.log(1.0 + jnp.exp(-jnp.abs(s)))
    o_ref[...] = jnp.broadcast_to(val, (1, 1))


_tc_finish = pl.pallas_call(
    _tc_finish_body,
    out_shape=jax.ShapeDtypeStruct((1, 1), jnp.float32),
)


def kernel(focus, context, embeddings):
    focus = focus.astype(jnp.int32)
    context = context.astype(jnp.int32)
    vocab, emb_d = embeddings.shape
    slot_w = 128
    emb_t = embeddings.T
    pack, split = _tc_pack_manual(emb_d, vocab, slot_w)
    vocab_al = vocab // 128 * 128
    emb_slots = pack(emb_t, embeddings[vocab_al:vocab, :])
    partials = _sc_partial_dot(
        emb_slots.shape[0], slot_w, focus.shape[0], emb_d, split)(
        focus, context, emb_slots)
    return _tc_finish(partials)


# speedup vs baseline: 1.0696x; 1.0092x over previous
import functools

import jax
import jax.numpy as jnp
from jax import lax
from jax.experimental import pallas as pl
from jax.experimental.pallas import tpu as pltpu
from jax.experimental.pallas import tpu_sc as plsc

_LANES = 16
_VB = 1024


def _pack_body(a_ref, b_ref, o_ref):
    emb_d = a_ref.shape[0]
    slot_w = o_ref.shape[1]
    di = lax.broadcasted_iota(jnp.int32, (emb_d, slot_w), 0)
    ji = lax.broadcasted_iota(jnp.int32, (emb_d, slot_w), 1)
    itop = (di == ji).astype(jnp.float32)
    ibot = (di + emb_d == ji).astype(jnp.float32)
    dn = (((0,), (0,)), ((), ()))
    o_ref[...] = (
        lax.dot_general(a_ref[...], itop, dn, preferred_element_type=jnp.float32)
        + lax.dot_general(b_ref[...], ibot, dn, preferred_element_type=jnp.float32)
    )


@functools.lru_cache(maxsize=None)
def _tc_pack_manual(emb_d, vocab, slot_w):
    split = (vocab // 2 + _VB - 1) // _VB * _VB
    nblk = split // _VB
    vocab_al = vocab // 128 * 128
    tail_n = vocab - vocab_al
    tail_lo = (vocab_al - split) - (nblk - 1) * _VB

    nbuf = 4

    def body(t_hbm, tail_ref, o_hbm, ab_buf, o_buf, sab, so):
        def bsize(g):
            return min(vocab_al - (split + g * _VB), _VB)

        def start_in(g, slot):
            pltpu.make_async_copy(
                t_hbm.at[:, pl.ds(g * _VB, _VB)],
                ab_buf.at[slot, pl.ds(0, emb_d), :], sab.at[slot],
            ).start()
            w = bsize(g)
            pltpu.make_async_copy(
                t_hbm.at[:, pl.ds(split + g * _VB, w)],
                ab_buf.at[slot, pl.ds(emb_d, emb_d), pl.ds(0, w)],
                sab.at[slot],
            ).start()

        def wait_in(g, slot):
            pltpu.make_async_copy(
                t_hbm.at[:, pl.ds(0, _VB)],
                ab_buf.at[slot, pl.ds(0, emb_d), :], sab.at[slot]).wait()
            w = bsize(g)
            pltpu.make_async_copy(
                t_hbm.at[:, pl.ds(0, w)],
                ab_buf.at[slot, pl.ds(0, emb_d), pl.ds(0, w)],
                sab.at[slot]).wait()

        def out_copy(g, slot):
            return pltpu.make_async_copy(
                o_buf.at[slot], o_hbm.at[pl.ds(g * _VB, _VB), :], so.at[slot])

        for g in range(min(nbuf, nblk)):
            start_in(g, g % nbuf)
        for g in range(nblk):
            slot = g % nbuf
            wait_in(g, slot)
            if g >= nbuf:
                out_copy(g - nbuf, slot).wait()
            o_buf[slot, :, pl.ds(0, emb_d)] = ab_buf[slot, pl.ds(0, emb_d), :].T
            o_buf[slot, :, pl.ds(emb_d, emb_d)] = (
                ab_buf[slot, pl.ds(emb_d, emb_d), :].T)
            if g == nblk - 1 and tail_n:
                o_buf[slot, pl.ds(tail_lo, tail_n), pl.ds(emb_d, emb_d)] = (
                    tail_ref[...])
            out_copy(g, slot).start()
            if g + nbuf < nblk:
                start_in(g + nbuf, slot)
        for g in range(max(nblk - nbuf, 0), nblk):
            out_copy(g, g % nbuf).wait()

    return pl.pallas_call(
        body,
        in_specs=[pl.BlockSpec(memory_space=pl.ANY), pl.BlockSpec()],
        out_specs=pl.BlockSpec(memory_space=pl.ANY),
        out_shape=jax.ShapeDtypeStruct((split, slot_w), jnp.float32),
        scratch_shapes=[
            pltpu.VMEM((nbuf, slot_w, _VB), jnp.float32),
            pltpu.VMEM((nbuf, _VB, slot_w), jnp.float32),
            pltpu.SemaphoreType.DMA((nbuf,)),
            pltpu.SemaphoreType.DMA((nbuf,)),
        ],
    ), split


@functools.lru_cache(maxsize=None)
def _tc_pack(emb_d, vocab, slot_w):
    split = (vocab // 2 + _VB - 1) // _VB * _VB
    grid = split // _VB
    nblk = split // _VB

    return pl.pallas_call(
        _pack_body,
        grid=(grid,),
        in_specs=[
            pl.BlockSpec((emb_d, _VB), lambda i: (0, i)),
            pl.BlockSpec((emb_d, _VB), lambda i, n=nblk: (0, i + n)),
        ],
        out_specs=pl.BlockSpec((_VB, slot_w), lambda i: (i, 0)),
        out_shape=jax.ShapeDtypeStruct((split, slot_w), jnp.float32),
        compiler_params=pltpu.CompilerParams(
            dimension_semantics=("parallel",)),
    ), split


@functools.lru_cache(maxsize=None)
def _sc_partial_dot(n_slots, slot_w, batch, emb_d, split):
    info = plsc.get_sparse_core_info()
    nc, ns = info.num_cores, info.num_subcores
    nw = nc * ns
    assert batch % nw == 0
    b_per_w = batch // nw
    assert b_per_w <= 128
    assert emb_d % _LANES == 0
    chunks = emb_d // _LANES

    mesh = plsc.VectorSubcoreMesh(core_axis_name="c", subcore_axis_name="s")

    @functools.partial(
        pl.kernel,
        out_type=jax.ShapeDtypeStruct((nw, _LANES), jnp.float32),
        mesh=mesh,
        scratch_types=[
            pltpu.VMEM((b_per_w,), jnp.int32),
            pltpu.VMEM((b_per_w,), jnp.int32),
            pltpu.VMEM((b_per_w,), jnp.int32),
            pltpu.VMEM((b_per_w,), jnp.int32),
            pltpu.VMEM((b_per_w, slot_w), jnp.float32),
            pltpu.VMEM((b_per_w, slot_w), jnp.float32),
            pltpu.VMEM((_LANES,), jnp.float32),
            pltpu.SemaphoreType.DMA,
        ],
    )
    def sc_kernel(focus_hbm, context_hbm, emb_hbm, out_hbm,
                  idx_f, idx_c, slot_f, slot_c, rows_f, rows_c, acc_v, sem):
        wid = lax.axis_index("s") * nc + lax.axis_index("c")
        base = wid * b_per_w
        pltpu.sync_copy(focus_hbm.at[pl.ds(base, b_per_w)], idx_f)
        pltpu.sync_copy(context_hbm.at[pl.ds(base, b_per_w)], idx_c)
        for k in range(b_per_w // _LANES):
            sl = pl.ds(k * _LANES, _LANES)
            vf = idx_f[sl]
            vc = idx_c[sl]
            slot_f[sl] = jnp.where(vf >= split, vf - split, vf)
            slot_c[sl] = jnp.where(vc >= split, vc - split, vc)
        cp_f = pltpu.async_copy(emb_hbm.at[slot_f], rows_f, sem)
        cp_c = pltpu.async_copy(emb_hbm.at[slot_c], rows_c, sem)
        cp_f.wait()
        cp_c.wait()

        def body(k, acc):
            base_k = k * _LANES
            vf = idx_f[pl.ds(base_k, _LANES)]
            vc = idx_c[pl.ds(base_k, _LANES)]
            pf = jnp.where(vf >= split, emb_d, 0)
            pc = jnp.where(vc >= split, emb_d, 0)
            for r in range(_LANES):
                i = base_k + r
                f_off = pf[r]
                c_off = pc[r]
                for j in range(chunks):
                    f = rows_f[i, pl.ds(f_off + j * _LANES, _LANES)]
                    c = rows_c[i, pl.ds(c_off + j * _LANES, _LANES)]
                    acc = acc + f * c
            return acc

        acc = lax.fori_loop(0, b_per_w // _LANES, body,
                            jnp.zeros((_LANES,), jnp.float32))
        acc_v[...] = acc
        pltpu.sync_copy(acc_v, out_hbm.at[wid])

    return sc_kernel


def _tc_finish_body(p_ref, o_ref):
    s = jnp.sum(p_ref[...])
    val = jnp.minimum(s, 0.0) - jnp.log(1.0 + jnp.exp(-jnp.abs(s)))
    o_ref[...] = jnp.broadcast_to(val, (1, 1))


_tc_finish = pl.pallas_call(
    _tc_finish_body,
    out_shape=jax.ShapeDtypeStruct((1, 1), jnp.float32),
)


def kernel(focus, context, embeddings):
    focus = focus.astype(jnp.int32)
    context = context.astype(jnp.int32)
    vocab, emb_d = embeddings.shape
    slot_w = 128
    emb_t = embeddings.T
    pack, split = _tc_pack_manual(emb_d, vocab, slot_w)
    vocab_al = vocab // 128 * 128
    emb_slots = pack(emb_t, embeddings[vocab_al:vocab, :])
    partials = _sc_partial_dot(
        emb_slots.shape[0], slot_w, focus.shape[0], emb_d, split)(
        focus, context, emb_slots)
    return _tc_finish(partials)
